# baseline (speedup 1.0000x reference)
import functools

import jax
import jax.numpy as jnp
from jax import lax
from jax.experimental import pallas as pl
from jax.experimental.pallas import tpu as pltpu
from jax.experimental.pallas import tpu_sc as plsc

_D = 64
_OUT = 100
_B = 4096
_L = 200
_W = 20

_NC = 2
_NS = 16
_NW = _NC * _NS
_BPW = _B // _NW
_CHUNKS = ((0, 128), (128, 72))
_LANES = 16
_DV = _D // _LANES


_LP = 256


def _sc_embed_sum(xp, table):
    mesh = plsc.VectorSubcoreMesh(
        core_axis_name="c", subcore_axis_name="s",
        num_cores=_NC, num_subcores=_NS)

    @functools.partial(
        pl.kernel,
        out_type=jax.ShapeDtypeStruct((_B, _D), jnp.float32),
        mesh=mesh,
        compiler_params=pltpu.CompilerParams(use_tc_tiling_on_sc=False),
        scratch_types=[
            pltpu.VMEM((_BPW, _LP), jnp.int32),
            pltpu.VMEM((2, _L, _D), jnp.float32),
            pltpu.VMEM((_BPW, _D), jnp.float32),
            pltpu.SemaphoreType.DMA,
            pltpu.SemaphoreType.DMA,
        ],
    )
    def body(x_hbm, tab_hbm, sent_hbm, idx_v, rows_v, sums_v, gsem0, gsem1):
        wid = lax.axis_index("s") * _NC + lax.axis_index("c")
        base = wid * _BPW
        pltpu.sync_copy(x_hbm.at[pl.ds(base, _BPW)], idx_v)
        sems = (gsem0, gsem1)

        def fire(i, slot):
            for off, sz in _CHUNKS:
                pltpu.async_copy(
                    tab_hbm.at[idx_v.at[i, pl.ds(off, sz)]],
                    rows_v.at[slot, pl.ds(off, sz)], sems[slot])

        def drain(slot):
            for off, sz in _CHUNKS:
                pltpu.make_async_copy(
                    tab_hbm.at[idx_v.at[0, pl.ds(off, sz)]],
                    rows_v.at[slot, pl.ds(off, sz)], sems[slot]).wait()

        fire(0, 0)
        fire(1, 1)

        @pl.loop(0, _BPW, step=2)
        def _rows(i):
            for b in range(2):
                ib = i + b
                drain(b)
                zero = jnp.zeros((_LANES,), jnp.float32)

                def red(c, carry):
                    return tuple(
                        carry[k] + rows_v[b, c, pl.ds(k * _LANES, _LANES)]
                        for k in range(_DV))

                acc = lax.fori_loop(0, _L, red, (zero,) * _DV, unroll=4)
                for k in range(_DV):
                    sums_v[ib, pl.ds(k * _LANES, _LANES)] = acc[k]

                @pl.when(ib + 2 < _BPW)
                def _():
                    fire(ib + 2, b)

        pltpu.sync_copy(sums_v, sent_hbm.at[pl.ds(base, _BPW)])

    return body(xp, table)


def _fc_kernel(s_ref, w_ref, b_ref, o_ref):
    o_ref[...] = (
        jnp.dot(s_ref[...], w_ref[...], preferred_element_type=jnp.float32)
        * (1.0 / _W)
        + b_ref[...]
    )


def _fc(sent, w_t, fc_b):
    return pl.pallas_call(
        _fc_kernel,
        out_shape=jax.ShapeDtypeStruct((_B, _OUT), jnp.float32),
    )(sent, w_t, fc_b[None, :])


def kernel(x, word_incices, table, fc_w, fc_b):
    del word_incices
    xp = jnp.pad(x, ((0, 0), (0, _LP - _L)))
    sent = _sc_embed_sum(xp, table)
    return _fc(sent, fc_w.T, fc_b)


# speedup vs baseline: 1.0008x; 1.0008x over previous
import functools

import jax
import jax.numpy as jnp
from jax import lax
from jax.experimental import pallas as pl
from jax.experimental.pallas import tpu as pltpu
from jax.experimental.pallas import tpu_sc as plsc

_D = 64
_OUT = 100
_B = 4096
_L = 200
_W = 20

_NC = 2
_NS = 16
_NW = _NC * _NS
_BPW = _B // _NW
_CHUNKS = ((0, 128), (128, 72))
_LANES = 16
_DV = _D // _LANES


_LP = 256


def _sc_embed_sum(xf, table):
    mesh = plsc.VectorSubcoreMesh(
        core_axis_name="c", subcore_axis_name="s",
        num_cores=_NC, num_subcores=_NS)

    @functools.partial(
        pl.kernel,
        out_type=jax.ShapeDtypeStruct((_B, _D), jnp.float32),
        mesh=mesh,
        compiler_params=pltpu.CompilerParams(use_tc_tiling_on_sc=False),
        scratch_types=[
            pltpu.VMEM((_BPW * _LP,), jnp.int32),
            pltpu.VMEM((2, _L, _D), jnp.float32),
            pltpu.VMEM((_BPW, _D), jnp.float32),
            pltpu.SemaphoreType.DMA,
            pltpu.SemaphoreType.DMA,
        ],
    )
    def body(x_hbm, tab_hbm, sent_hbm, idx_v, rows_v, sums_v, gsem0, gsem1):
        wid = lax.axis_index("s") * _NC + lax.axis_index("c")
        base = wid * _BPW
        pltpu.sync_copy(x_hbm.at[pl.ds(base * _LP, _BPW * _LP)], idx_v)
        sems = (gsem0, gsem1)

        def fire(i, slot):
            for off, sz in _CHUNKS:
                pltpu.async_copy(
                    tab_hbm.at[idx_v.at[pl.ds(i * _LP + off, sz)]],
                    rows_v.at[slot, pl.ds(off, sz)], sems[slot])

        def drain(slot):
            for off, sz in _CHUNKS:
                pltpu.make_async_copy(
                    tab_hbm.at[idx_v.at[pl.ds(off, sz)]],
                    rows_v.at[slot, pl.ds(off, sz)], sems[slot]).wait()

        fire(0, 0)
        fire(1, 1)

        @pl.loop(0, _BPW, step=2)
        def _rows(i):
            for b in range(2):
                ib = i + b
                drain(b)
                zero = jnp.zeros((_LANES,), jnp.float32)

                def red(c, carry):
                    return tuple(
                        carry[k] + rows_v[b, c, pl.ds(k * _LANES, _LANES)]
                        for k in range(_DV))

                acc = lax.fori_loop(0, _L, red, (zero,) * _DV, unroll=4)
                for k in range(_DV):
                    sums_v[ib, pl.ds(k * _LANES, _LANES)] = acc[k]

                @pl.when(ib + 2 < _BPW)
                def _():
                    fire(ib + 2, b)

        pltpu.sync_copy(sums_v, sent_hbm.at[pl.ds(base, _BPW)])

    return body(xf, table)


def _fc_kernel(s_ref, w_ref, b_ref, o_ref):
    o_ref[...] = (
        jnp.dot(s_ref[...], w_ref[...], preferred_element_type=jnp.float32)
        * (1.0 / _W)
        + b_ref[...]
    )


def _fc(sent, w_t, fc_b):
    return pl.pallas_call(
        _fc_kernel,
        out_shape=jax.ShapeDtypeStruct((_B, _OUT), jnp.float32),
    )(sent, w_t, fc_b[None, :])


def kernel(x, word_incices, table, fc_w, fc_b):
    del word_incices
    xf = jnp.pad(x, ((0, 0), (0, _LP - _L))).reshape(-1)
    sent = _sc_embed_sum(xf, table)
    return _fc(sent, fc_w.T, fc_b)
